# Initial kernel scaffold; baseline (speedup 1.0000x reference)
import jax
import jax.numpy as jnp
from jax.experimental import pallas as pl


def kernel(events, features, lengths, router_W, router_b, kan_W1, kan_b1, kan_W2, kan_b2, feat_W, mamba_Win, mamba_a, mamba_b, mamba_c, mamba_d, lstm_Wih1, lstm_Whh1, lstm_bih1, lstm_bhh1, lstm_Wih2, lstm_Whh2, lstm_bih2, lstm_bhh2, cls_W, cls_b):
    raise NotImplementedError("write your pallas kernel here")


# speedup vs baseline: 11.1015x; 11.1015x over previous
import functools

import jax
import jax.numpy as jnp
from jax import lax
from jax.experimental import pallas as pl
from jax.experimental.pallas import tpu as pltpu
from jax.experimental.pallas import tpu_sc as plsc

B, L, D, E, G, AUX, H, NCLS = 256, 200, 32, 4, 5, 16, 128, 10
GAMMA = 0.3
NV = 784


def _table_body(rw_ref, rb_ref, w1_ref, b1_ref, bd2_ref, b2_ref, win_ref,
                tab_ref):
    t = lax.broadcasted_iota(jnp.int32, (NV, 1), 0).astype(jnp.float32) * (
        1.0 / 784.0)
    rw = rw_ref[...]
    rb = rb_ref[...]
    ls = [t * rw[:, e:e + 1] + rb[:, e:e + 1] for e in range(E)]

    sel = []
    for e in range(E):
        cnt = jnp.zeros_like(ls[e])
        for j in range(E):
            if j == e:
                continue
            beats = (ls[j] > ls[e]) if j > e else (ls[j] >= ls[e])
            cnt = cnt + jnp.where(beats, 1.0, 0.0)
        sel.append(cnt < 2.0)
    m = jnp.maximum(jnp.maximum(ls[0], ls[1]), jnp.maximum(ls[2], ls[3]))
    we = [jnp.where(sel[e], jnp.exp(ls[e] - m), 0.0) for e in range(E)]
    den = we[0] + we[1] + we[2] + we[3]
    wn = [w / den for w in we]

    acc = jnp.broadcast_to(b1_ref[...], (NV, E * 32))
    w1 = w1_ref[...]
    for g in range(G):
        gv = -2.0 + float(g)
        bg = jnp.exp(-(((t - gv) * 2.0) ** 2))
        acc = acc + bg * w1[g:g + 1, :]
    h1 = acc * jax.nn.sigmoid(acc)
    all_out = jnp.dot(h1, bd2_ref[...],
                      preferred_element_type=jnp.float32) + b2_ref[...]
    eo = jnp.zeros((NV, D), jnp.float32)
    for e in range(E):
        eo = eo + wn[e] * all_out[:, e * D:(e + 1) * D]
    xm = jnp.dot(eo, win_ref[...], preferred_element_type=jnp.float32)
    tab_ref[...] = jnp.concatenate(
        [eo, xm, jnp.zeros((NV, 2 * D), jnp.float32)], axis=1)


def _build_tables(router_W, router_b, kan_W1, kan_b1, kan_W2, kan_b2,
                  mamba_Win):
    w1r = kan_W1.transpose(1, 0, 2).reshape(G, E * 32)
    b1r = kan_b1.reshape(1, E * 32)
    bd2 = jax.scipy.linalg.block_diag(*[kan_W2[e] for e in range(E)])
    b2r = kan_b2.reshape(1, E * D)
    return pl.pallas_call(
        _table_body,
        out_shape=jax.ShapeDtypeStruct((NV, 4 * D), jnp.float32),
    )(router_W, router_b.reshape(1, E), w1r, b1r, bd2, b2r, mamba_Win)


_NW = 32
_BP = (B * L) // _NW
_CH = 80
_NCH = _BP // _CH


def _gather_body(idx_hbm, tab_hbm, g_hbm, idx_v, rows_v, sem):
    wid = lax.axis_index("s") * 2 + lax.axis_index("c")
    base = wid * _BP
    pltpu.sync_copy(idx_hbm.at[pl.ds(base, _BP)], idx_v)

    def chunk(i, carry):
        off = i * _CH
        pltpu.async_copy(tab_hbm.at[idx_v.at[pl.ds(off, _CH)]], rows_v,
                         sem).wait()
        pltpu.sync_copy(rows_v, g_hbm.at[pl.ds(base + off, _CH)])
        return carry

    lax.fori_loop(0, _NCH, chunk, 0)


def _gather(idx, table):
    mesh = plsc.VectorSubcoreMesh(core_axis_name="c", subcore_axis_name="s")
    f = functools.partial(
        pl.kernel,
        mesh=mesh,
        out_type=jax.ShapeDtypeStruct((B * L, 4 * D), jnp.float32),
        scratch_types=[pltpu.VMEM((_BP,), jnp.int32),
                       pltpu.VMEM((_CH, 4 * D), jnp.float32),
                       pltpu.SemaphoreType.DMA],
    )(_gather_body)
    return f(idx, table)


def _main_body(g_ref, feat_ref, len_ref,
               featw_ref, ma_ref, mb_ref, mc_ref, md_ref, e_ref,
               wih1_ref, whh1_ref, b1_ref, wih2_ref, whh2_ref, b2_ref,
               clsw_ref, clsb_ref, out_ref):
    featw = featw_ref[...]
    a_sig = jax.nn.sigmoid(ma_ref[...])
    mb = mb_ref[...]
    mc = mc_ref[...]
    md = md_ref[...]
    ev = e_ref[...]
    wih1 = wih1_ref[...]
    whh1 = whh1_ref[...]
    b1 = b1_ref[...]
    wih2 = wih2_ref[...]
    whh2 = whh2_ref[...]
    b2 = b2_ref[...]
    lens_raw = len_ref[...]
    lens = jnp.maximum(lens_raw, 1)

    def step(t, carry):
        s, h1, c1, h2, c2 = carry
        g = g_ref[t]
        ft = feat_ref[t]
        x = g + jnp.dot(ft, featw, preferred_element_type=jnp.float32)
        s = a_sig * s + mb * x
        y = mc * s + md * x
        emb = GAMMA * y + ev * g
        g1 = (jnp.dot(emb, wih1, preferred_element_type=jnp.float32)
              + jnp.dot(h1, whh1, preferred_element_type=jnp.float32) + b1)
        i1 = jax.nn.sigmoid(g1[:, 0:H])
        f1 = jax.nn.sigmoid(g1[:, H:2 * H])
        gg1 = jnp.tanh(g1[:, 2 * H:3 * H])
        o1 = jax.nn.sigmoid(g1[:, 3 * H:4 * H])
        c1n = f1 * c1 + i1 * gg1
        h1n = o1 * jnp.tanh(c1n)
        m = lens > t
        h1 = jnp.where(m, h1n, h1)
        c1 = jnp.where(m, c1n, c1)
        g2 = (jnp.dot(h1, wih2, preferred_element_type=jnp.float32)
              + jnp.dot(h2, whh2, preferred_element_type=jnp.float32) + b2)
        i2 = jax.nn.sigmoid(g2[:, 0:H])
        f2 = jax.nn.sigmoid(g2[:, H:2 * H])
        gg2 = jnp.tanh(g2[:, 2 * H:3 * H])
        o2 = jax.nn.sigmoid(g2[:, 3 * H:4 * H])
        c2n = f2 * c2 + i2 * gg2
        h2n = o2 * jnp.tanh(c2n)
        h2 = jnp.where(m, h2n, h2)
        c2 = jnp.where(m, c2n, c2)
        return (s, h1, c1, h2, c2)

    z32 = jnp.zeros((B, 4 * D), jnp.float32)
    zh = jnp.zeros((B, H), jnp.float32)
    _, _, _, h2, _ = lax.fori_loop(0, L, step, (z32, zh, zh, zh, zh))
    logits = jnp.dot(h2, clsw_ref[...],
                     preferred_element_type=jnp.float32) + clsb_ref[...]
    valid = lens_raw > 0
    out_ref[...] = jnp.where(valid, logits, 0.0)


def _main(g3, feat_t, lengths, featw_p, ma_p, mb_p, mc_p, md_p, ev_p,
          wih1_s, lstm_Whh1, bsum1, lstm_Wih2, lstm_Whh2, bsum2,
          cls_W, cls_b):
    return pl.pallas_call(
        _main_body,
        out_shape=jax.ShapeDtypeStruct((B, NCLS), jnp.float32),
        compiler_params=pltpu.CompilerParams(
            vmem_limit_bytes=100 * 1024 * 1024),
    )(g3, feat_t, lengths, featw_p, ma_p, mb_p, mc_p, md_p, ev_p,
      wih1_s, lstm_Whh1, bsum1, lstm_Wih2, lstm_Whh2, bsum2,
      cls_W, cls_b.reshape(1, NCLS))


def kernel(events, features, lengths, router_W, router_b, kan_W1, kan_b1,
           kan_W2, kan_b2, feat_W, mamba_Win, mamba_a, mamba_b, mamba_c,
           mamba_d, lstm_Wih1, lstm_Whh1, lstm_bih1, lstm_bhh1,
           lstm_Wih2, lstm_Whh2, lstm_bih2, lstm_bhh2, cls_W, cls_b):
    table = _build_tables(router_W, router_b, kan_W1, kan_b1,
                          kan_W2, kan_b2, mamba_Win)
    idx = events.astype(jnp.int32).T.reshape(B * L)
    g3 = _gather(idx, table).reshape(L, B, 4 * D)
    feat_t = features.transpose(1, 0, 2)
    zc = jnp.zeros((1, D), jnp.float32)
    pad = lambda v: jnp.concatenate(
        [zc, v.reshape(1, D), zc, zc], axis=1)
    featw_p = jnp.concatenate(
        [jnp.zeros((AUX, D), jnp.float32), feat_W,
         jnp.zeros((AUX, 2 * D), jnp.float32)], axis=1)
    ev_p = jnp.concatenate(
        [jnp.full((1, D), 1.0 - GAMMA, jnp.float32), zc, zc, zc], axis=1)
    wih1_s = jnp.concatenate(
        [lstm_Wih1, lstm_Wih1, jnp.zeros((2 * D, 4 * H), jnp.float32)],
        axis=0)
    bsum1 = (lstm_bih1 + lstm_bhh1).reshape(1, 4 * H)
    bsum2 = (lstm_bih2 + lstm_bhh2).reshape(1, 4 * H)
    return _main(g3, feat_t, lengths.reshape(B, 1).astype(jnp.int32),
                 featw_p, pad(mamba_a), pad(mamba_b), pad(mamba_c),
                 pad(mamba_d), ev_p, wih1_s, lstm_Whh1, bsum1,
                 lstm_Wih2, lstm_Whh2, bsum2, cls_W, cls_b)


# speedup vs baseline: 13.7659x; 1.2400x over previous
import functools

import jax
import jax.numpy as jnp
from jax import lax
from jax.experimental import pallas as pl
from jax.experimental.pallas import tpu as pltpu
from jax.experimental.pallas import tpu_sc as plsc

B, L, D, E, G, AUX, H, NCLS = 256, 200, 32, 4, 5, 16, 128, 10
GAMMA = 0.3
NV = 784


def _table_body(rw_ref, rb_ref, w1_ref, b1_ref, bd2_ref, b2_ref, win_ref,
                tab_ref):
    t = lax.broadcasted_iota(jnp.int32, (NV, 1), 0).astype(jnp.float32) * (
        1.0 / 784.0)
    rw = rw_ref[...]
    rb = rb_ref[...]
    ls = [t * rw[:, e:e + 1] + rb[:, e:e + 1] for e in range(E)]

    sel = []
    for e in range(E):
        cnt = jnp.zeros_like(ls[e])
        for j in range(E):
            if j == e:
                continue
            beats = (ls[j] > ls[e]) if j > e else (ls[j] >= ls[e])
            cnt = cnt + jnp.where(beats, 1.0, 0.0)
        sel.append(cnt < 2.0)
    m = jnp.maximum(jnp.maximum(ls[0], ls[1]), jnp.maximum(ls[2], ls[3]))
    we = [jnp.where(sel[e], jnp.exp(ls[e] - m), 0.0) for e in range(E)]
    den = we[0] + we[1] + we[2] + we[3]
    wn = [w / den for w in we]

    acc = jnp.broadcast_to(b1_ref[...], (NV, E * 32))
    w1 = w1_ref[...]
    for g in range(G):
        gv = -2.0 + float(g)
        bg = jnp.exp(-(((t - gv) * 2.0) ** 2))
        acc = acc + bg * w1[g:g + 1, :]
    h1 = acc * jax.nn.sigmoid(acc)
    all_out = jnp.dot(h1, bd2_ref[...],
                      preferred_element_type=jnp.float32) + b2_ref[...]
    eo = jnp.zeros((NV, D), jnp.float32)
    for e in range(E):
        eo = eo + wn[e] * all_out[:, e * D:(e + 1) * D]
    xm = jnp.dot(eo, win_ref[...], preferred_element_type=jnp.float32)
    tab_ref[...] = jnp.concatenate(
        [eo, xm, jnp.zeros((NV, 2 * D), jnp.float32)], axis=1)


def _build_tables(router_W, router_b, kan_W1, kan_b1, kan_W2, kan_b2,
                  mamba_Win):
    w1r = kan_W1.transpose(1, 0, 2).reshape(G, E * 32)
    b1r = kan_b1.reshape(1, E * 32)
    bd2 = jax.scipy.linalg.block_diag(*[kan_W2[e] for e in range(E)])
    b2r = kan_b2.reshape(1, E * D)
    return pl.pallas_call(
        _table_body,
        out_shape=jax.ShapeDtypeStruct((NV, 4 * D), jnp.float32),
    )(router_W, router_b.reshape(1, E), w1r, b1r, bd2, b2r, mamba_Win)


_NW = 32
_BP = (B * L) // _NW
_CH = 80
_NCH = _BP // _CH


_GRP = 4


def _gather_body(idx_hbm, tab_hbm, g_hbm, idx_v, rows_v, sem_g, sem_o):
    wid = lax.axis_index("s") * 2 + lax.axis_index("c")
    base = wid * _BP
    pltpu.sync_copy(idx_hbm.at[pl.ds(base, _BP)], idx_v)

    def group(grp, carry):
        offs = [grp * _GRP * _CH + j * _CH for j in range(_GRP)]
        bufs = [rows_v.at[j] for j in range(_GRP)]
        @pl.when(grp > 0)
        def _():
            for j in range(_GRP):
                pltpu.make_async_copy(
                    bufs[j], g_hbm.at[pl.ds(base + offs[j], _CH)],
                    sem_o).wait()
        for j in range(_GRP):
            pltpu.async_copy(tab_hbm.at[idx_v.at[pl.ds(offs[j], _CH)]],
                             bufs[j], sem_g)
        for j in range(_GRP):
            pltpu.make_async_copy(
                tab_hbm.at[idx_v.at[pl.ds(offs[j], _CH)]], bufs[j],
                sem_g).wait()
        for j in range(_GRP):
            pltpu.async_copy(bufs[j], g_hbm.at[pl.ds(base + offs[j], _CH)],
                             sem_o)
        return carry

    lax.fori_loop(0, _NCH // _GRP, group, 0)
    for j in range(_GRP):
        pltpu.make_async_copy(rows_v.at[j],
                              g_hbm.at[pl.ds(base, _CH)], sem_o).wait()


def _gather(idx, table):
    mesh = plsc.VectorSubcoreMesh(core_axis_name="c", subcore_axis_name="s")
    f = functools.partial(
        pl.kernel,
        mesh=mesh,
        out_type=jax.ShapeDtypeStruct((B * L, 4 * D), jnp.float32),
        scratch_types=[pltpu.VMEM((_BP,), jnp.int32),
                       pltpu.VMEM((_GRP, _CH, 4 * D), jnp.float32),
                       pltpu.SemaphoreType.DMA,
                       pltpu.SemaphoreType.DMA],
    )(_gather_body)
    return f(idx, table)


def _main_body(g_ref, feat_ref, len_ref,
               featw_ref, ma_ref, mb_ref, mc_ref, md_ref, e_ref,
               wih1_ref, whh1_ref, b1_ref, wih2_ref, whh2_ref, b2_ref,
               clsw_ref, clsb_ref, out_ref):
    featw = featw_ref[...]
    a_sig = jax.nn.sigmoid(ma_ref[...])
    mb = mb_ref[...]
    mc = mc_ref[...]
    md = md_ref[...]
    ev = e_ref[...]
    wih1 = wih1_ref[...]
    whh1 = whh1_ref[...]
    b1 = b1_ref[...]
    wih2 = wih2_ref[...]
    whh2 = whh2_ref[...]
    b2 = b2_ref[...]
    lens_raw = len_ref[...]
    lens = jnp.maximum(lens_raw, 1)

    def step(t, carry):
        s, h1, c1, h2, c2 = carry
        g = g_ref[t]
        ft = feat_ref[t]
        x = g + jnp.dot(ft, featw, preferred_element_type=jnp.float32)
        s = a_sig * s + mb * x
        y = mc * s + md * x
        emb = GAMMA * y + ev * g
        g1 = (jnp.dot(emb, wih1, preferred_element_type=jnp.float32)
              + jnp.dot(h1, whh1, preferred_element_type=jnp.float32) + b1)
        i1 = jax.nn.sigmoid(g1[:, 0:H])
        f1 = jax.nn.sigmoid(g1[:, H:2 * H])
        gg1 = jnp.tanh(g1[:, 2 * H:3 * H])
        o1 = jax.nn.sigmoid(g1[:, 3 * H:4 * H])
        c1n = f1 * c1 + i1 * gg1
        h1n = o1 * jnp.tanh(c1n)
        m = lens > t
        h1 = jnp.where(m, h1n, h1)
        c1 = jnp.where(m, c1n, c1)
        g2 = (jnp.dot(h1, wih2, preferred_element_type=jnp.float32)
              + jnp.dot(h2, whh2, preferred_element_type=jnp.float32) + b2)
        i2 = jax.nn.sigmoid(g2[:, 0:H])
        f2 = jax.nn.sigmoid(g2[:, H:2 * H])
        gg2 = jnp.tanh(g2[:, 2 * H:3 * H])
        o2 = jax.nn.sigmoid(g2[:, 3 * H:4 * H])
        c2n = f2 * c2 + i2 * gg2
        h2n = o2 * jnp.tanh(c2n)
        h2 = jnp.where(m, h2n, h2)
        c2 = jnp.where(m, c2n, c2)
        return (s, h1, c1, h2, c2)

    def step4(i, carry):
        for k in range(4):
            carry = step(4 * i + k, carry)
        return carry

    z32 = jnp.zeros((B, 4 * D), jnp.float32)
    zh = jnp.zeros((B, H), jnp.float32)
    _, _, _, h2, _ = lax.fori_loop(0, L // 4, step4, (z32, zh, zh, zh, zh))
    logits = jnp.dot(h2, clsw_ref[...],
                     preferred_element_type=jnp.float32) + clsb_ref[...]
    valid = lens_raw > 0
    out_ref[...] = jnp.where(valid, logits, 0.0)


def _main(g3, feat_t, lengths, featw_p, ma_p, mb_p, mc_p, md_p, ev_p,
          wih1_s, lstm_Whh1, bsum1, lstm_Wih2, lstm_Whh2, bsum2,
          cls_W, cls_b):
    return pl.pallas_call(
        _main_body,
        out_shape=jax.ShapeDtypeStruct((B, NCLS), jnp.float32),
        compiler_params=pltpu.CompilerParams(
            vmem_limit_bytes=100 * 1024 * 1024),
    )(g3, feat_t, lengths, featw_p, ma_p, mb_p, mc_p, md_p, ev_p,
      wih1_s, lstm_Whh1, bsum1, lstm_Wih2, lstm_Whh2, bsum2,
      cls_W, cls_b.reshape(1, NCLS))


def kernel(events, features, lengths, router_W, router_b, kan_W1, kan_b1,
           kan_W2, kan_b2, feat_W, mamba_Win, mamba_a, mamba_b, mamba_c,
           mamba_d, lstm_Wih1, lstm_Whh1, lstm_bih1, lstm_bhh1,
           lstm_Wih2, lstm_Whh2, lstm_bih2, lstm_bhh2, cls_W, cls_b):
    table = _build_tables(router_W, router_b, kan_W1, kan_b1,
                          kan_W2, kan_b2, mamba_Win)
    idx = events.astype(jnp.int32).T.reshape(B * L)
    g3 = _gather(idx, table).reshape(L, B, 4 * D)
    feat_t = features.transpose(1, 0, 2)
    zc = jnp.zeros((1, D), jnp.float32)
    pad = lambda v: jnp.concatenate(
        [zc, v.reshape(1, D), zc, zc], axis=1)
    featw_p = jnp.concatenate(
        [jnp.zeros((AUX, D), jnp.float32), feat_W,
         jnp.zeros((AUX, 2 * D), jnp.float32)], axis=1)
    ev_p = jnp.concatenate(
        [jnp.full((1, D), 1.0 - GAMMA, jnp.float32), zc, zc, zc], axis=1)
    wih1_s = jnp.concatenate(
        [lstm_Wih1, lstm_Wih1, jnp.zeros((2 * D, 4 * H), jnp.float32)],
        axis=0)
    bsum1 = (lstm_bih1 + lstm_bhh1).reshape(1, 4 * H)
    bsum2 = (lstm_bih2 + lstm_bhh2).reshape(1, 4 * H)
    return _main(g3, feat_t, lengths.reshape(B, 1).astype(jnp.int32),
                 featw_p, pad(mamba_a), pad(mamba_b), pad(mamba_c),
                 pad(mamba_d), ev_p, wih1_s, lstm_Whh1, bsum1,
                 lstm_Wih2, lstm_Whh2, bsum2, cls_W, cls_b)


# speedup vs baseline: 16.8585x; 1.2247x over previous
import functools

import jax
import jax.numpy as jnp
from jax import lax
from jax.experimental import pallas as pl
from jax.experimental.pallas import tpu as pltpu
from jax.experimental.pallas import tpu_sc as plsc

B, L, D, E, G, AUX, H, NCLS = 256, 200, 32, 4, 5, 16, 128, 10
GAMMA = 0.3
NV = 784


def _table_body(rw_ref, rb_ref, w1_ref, b1_ref, bd2_ref, b2_ref, win_ref,
                tab_ref):
    t = lax.broadcasted_iota(jnp.int32, (NV, 1), 0).astype(jnp.float32) * (
        1.0 / 784.0)
    rw = rw_ref[...]
    rb = rb_ref[...]
    ls = [t * rw[:, e:e + 1] + rb[:, e:e + 1] for e in range(E)]

    sel = []
    for e in range(E):
        cnt = jnp.zeros_like(ls[e])
        for j in range(E):
            if j == e:
                continue
            beats = (ls[j] > ls[e]) if j > e else (ls[j] >= ls[e])
            cnt = cnt + jnp.where(beats, 1.0, 0.0)
        sel.append(cnt < 2.0)
    m = jnp.maximum(jnp.maximum(ls[0], ls[1]), jnp.maximum(ls[2], ls[3]))
    we = [jnp.where(sel[e], jnp.exp(ls[e] - m), 0.0) for e in range(E)]
    den = we[0] + we[1] + we[2] + we[3]
    wn = [w / den for w in we]

    acc = jnp.broadcast_to(b1_ref[...], (NV, E * 32))
    w1 = w1_ref[...]
    for g in range(G):
        gv = -2.0 + float(g)
        bg = jnp.exp(-(((t - gv) * 2.0) ** 2))
        acc = acc + bg * w1[g:g + 1, :]
    h1 = acc * jax.nn.sigmoid(acc)
    all_out = jnp.dot(h1, bd2_ref[...],
                      preferred_element_type=jnp.float32) + b2_ref[...]
    eo = jnp.zeros((NV, D), jnp.float32)
    for e in range(E):
        eo = eo + wn[e] * all_out[:, e * D:(e + 1) * D]
    xm = jnp.dot(eo, win_ref[...], preferred_element_type=jnp.float32)
    tab_ref[...] = jnp.concatenate(
        [eo, xm, jnp.zeros((NV, 2 * D), jnp.float32)], axis=1)


def _build_tables(router_W, router_b, kan_W1, kan_b1, kan_W2, kan_b2,
                  mamba_Win):
    w1r = kan_W1.transpose(1, 0, 2).reshape(G, E * 32)
    b1r = kan_b1.reshape(1, E * 32)
    bd2 = jax.scipy.linalg.block_diag(*[kan_W2[e] for e in range(E)])
    b2r = kan_b2.reshape(1, E * D)
    return pl.pallas_call(
        _table_body,
        out_shape=jax.ShapeDtypeStruct((NV, 4 * D), jnp.float32),
    )(router_W, router_b.reshape(1, E), w1r, b1r, bd2, b2r, mamba_Win)


_NW = 32
_BP = (B * L) // _NW
_CH = 80
_NCH = _BP // _CH


_GRP = 10


def _gather_body(idx_hbm, tab_hbm, g_hbm, idx_v, rows_v, sem_g, sem_o):
    wid = lax.axis_index("s") * 2 + lax.axis_index("c")
    base = wid * _BP
    pltpu.sync_copy(idx_hbm.at[pl.ds(base, _BP)], idx_v)

    def group(grp, carry):
        offs = [grp * _GRP * _CH + j * _CH for j in range(_GRP)]
        bufs = [rows_v.at[j] for j in range(_GRP)]
        @pl.when(grp > 0)
        def _():
            for j in range(_GRP):
                pltpu.make_async_copy(
                    bufs[j], g_hbm.at[pl.ds(base + offs[j], _CH)],
                    sem_o).wait()
        for j in range(_GRP):
            pltpu.async_copy(tab_hbm.at[idx_v.at[pl.ds(offs[j], _CH)]],
                             bufs[j], sem_g)
        for j in range(_GRP):
            pltpu.make_async_copy(
                tab_hbm.at[idx_v.at[pl.ds(offs[j], _CH)]], bufs[j],
                sem_g).wait()
        for j in range(_GRP):
            pltpu.async_copy(bufs[j], g_hbm.at[pl.ds(base + offs[j], _CH)],
                             sem_o)
        return carry

    lax.fori_loop(0, _NCH // _GRP, group, 0)
    for j in range(_GRP):
        pltpu.make_async_copy(rows_v.at[j],
                              g_hbm.at[pl.ds(base, _CH)], sem_o).wait()


def _gather(idx, table):
    mesh = plsc.VectorSubcoreMesh(core_axis_name="c", subcore_axis_name="s")
    f = functools.partial(
        pl.kernel,
        mesh=mesh,
        out_type=jax.ShapeDtypeStruct((B * L, 4 * D), jnp.float32),
        scratch_types=[pltpu.VMEM((_BP,), jnp.int32),
                       pltpu.VMEM((_GRP, _CH, 4 * D), jnp.float32),
                       pltpu.SemaphoreType.DMA,
                       pltpu.SemaphoreType.DMA],
    )(_gather_body)
    return f(idx, table)


def _sig(x):
    return 0.5 * jnp.tanh(0.5 * x) + 0.5


_CS = 8
_NC = L // _CS


def _main_body(g_ref, feat_ref, len_ref,
               featw_ref, ma_ref, mb_ref, mc_ref, md_ref, e_ref,
               wih1_ref, whh1_ref, b1_ref, wih2_ref, whh2_ref, b2_ref,
               clsw_ref, clsb_ref, out_ref, gbuf, fbuf, semg, semf):
    def g_copy(i, slot):
        return pltpu.make_async_copy(
            g_ref.at[pl.ds(_CS * i, _CS)], gbuf.at[slot], semg.at[slot])

    def f_copy(i, slot):
        return pltpu.make_async_copy(
            feat_ref.at[pl.ds(_CS * AUX * i, _CS * AUX)], fbuf.at[slot],
            semf.at[slot])

    featw = featw_ref[...]
    a_sig = jax.nn.sigmoid(ma_ref[...])
    mb = mb_ref[...]
    mc = mc_ref[...]
    md = md_ref[...]
    ev = e_ref[...]
    wih1 = wih1_ref[...]
    whh1 = whh1_ref[...]
    b1 = b1_ref[...]
    wih2 = wih2_ref[...]
    whh2 = whh2_ref[...]
    b2 = b2_ref[...]
    lens_raw = len_ref[...]
    lens = jnp.maximum(lens_raw, 1)

    def step(t, gv, ftT, carry):
        s, h1, c1, h2, c2 = carry
        g = gv[:, 0:2 * D]
        x = g + lax.dot_general(
            ftT, featw, (((0,), (0,)), ((), ())),
            preferred_element_type=jnp.float32)
        s = a_sig * s + mb * x
        y = mc * s + md * x
        emb = GAMMA * y + ev * g
        g1 = (jnp.dot(emb, wih1, preferred_element_type=jnp.float32)
              + jnp.dot(h1, whh1, preferred_element_type=jnp.float32) + b1)
        i1 = _sig(g1[:, 0:H])
        f1 = _sig(g1[:, H:2 * H])
        gg1 = jnp.tanh(g1[:, 2 * H:3 * H])
        o1 = _sig(g1[:, 3 * H:4 * H])
        c1n = f1 * c1 + i1 * gg1
        h1n = o1 * jnp.tanh(c1n)
        m = lens > t
        h1 = jnp.where(m, h1n, h1)
        c1 = jnp.where(m, c1n, c1)
        g2 = (jnp.dot(h1, wih2, preferred_element_type=jnp.float32)
              + jnp.dot(h2, whh2, preferred_element_type=jnp.float32) + b2)
        i2 = _sig(g2[:, 0:H])
        f2 = _sig(g2[:, H:2 * H])
        gg2 = jnp.tanh(g2[:, 2 * H:3 * H])
        o2 = _sig(g2[:, 3 * H:4 * H])
        c2n = f2 * c2 + i2 * gg2
        h2n = o2 * jnp.tanh(c2n)
        h2 = jnp.where(m, h2n, h2)
        c2 = jnp.where(m, c2n, c2)
        return (s, h1, c1, h2, c2)

    def chunk(i, carry):
        slot = lax.rem(i, 2)

        @pl.when(i + 1 < _NC)
        def _():
            g_copy(i + 1, 1 - slot).start()
            f_copy(i + 1, 1 - slot).start()

        g_copy(i, slot).wait()
        f_copy(i, slot).wait()
        for k in range(_CS):
            gv = gbuf[slot, k]
            ftT = fbuf[slot, pl.ds(AUX * k, AUX), :]
            carry = step(_CS * i + k, gv, ftT, carry)
        return carry

    g_copy(0, 0).start()
    f_copy(0, 0).start()
    z32 = jnp.zeros((B, 2 * D), jnp.float32)
    zh = jnp.zeros((B, H), jnp.float32)
    _, _, _, h2, _ = lax.fori_loop(0, _NC, chunk, (z32, zh, zh, zh, zh))
    logits = jnp.dot(h2, clsw_ref[...],
                     preferred_element_type=jnp.float32) + clsb_ref[...]
    valid = lens_raw > 0
    out_ref[...] = jnp.where(valid, logits, 0.0)


def _main(g3, feat_t, lengths, featw_p, ma_p, mb_p, mc_p, md_p, ev_p,
          wih1_s, lstm_Whh1, bsum1, lstm_Wih2, lstm_Whh2, bsum2,
          cls_W, cls_b):
    return pl.pallas_call(
        _main_body,
        out_shape=jax.ShapeDtypeStruct((B, NCLS), jnp.float32),
        in_specs=[pl.BlockSpec(memory_space=pl.ANY)] * 2
        + [pl.BlockSpec(memory_space=pltpu.MemorySpace.VMEM)] * 15,
        scratch_shapes=[
            pltpu.VMEM((2, _CS, B, 4 * D), jnp.float32),
            pltpu.VMEM((2, _CS * AUX, B), jnp.float32),
            pltpu.SemaphoreType.DMA((2,)),
            pltpu.SemaphoreType.DMA((2,)),
        ],
        compiler_params=pltpu.CompilerParams(
            vmem_limit_bytes=100 * 1024 * 1024),
    )(g3, feat_t, lengths, featw_p, ma_p, mb_p, mc_p, md_p, ev_p,
      wih1_s, lstm_Whh1, bsum1, lstm_Wih2, lstm_Whh2, bsum2,
      cls_W, cls_b.reshape(1, NCLS))


def kernel(events, features, lengths, router_W, router_b, kan_W1, kan_b1,
           kan_W2, kan_b2, feat_W, mamba_Win, mamba_a, mamba_b, mamba_c,
           mamba_d, lstm_Wih1, lstm_Whh1, lstm_bih1, lstm_bhh1,
           lstm_Wih2, lstm_Whh2, lstm_bih2, lstm_bhh2, cls_W, cls_b):
    table = _build_tables(router_W, router_b, kan_W1, kan_b1,
                          kan_W2, kan_b2, mamba_Win)
    idx = events.astype(jnp.int32).T.reshape(B * L)
    g3 = _gather(idx, table).reshape(L, B, 4 * D)
    feat_t = features.reshape(B, L * AUX).T
    zc = jnp.zeros((1, D), jnp.float32)
    pad = lambda v: jnp.concatenate([zc, v.reshape(1, D)], axis=1)
    featw_p = jnp.concatenate(
        [jnp.zeros((AUX, D), jnp.float32), feat_W], axis=1)
    ev_p = jnp.concatenate(
        [jnp.full((1, D), 1.0 - GAMMA, jnp.float32), zc], axis=1)
    wih1_s = jnp.concatenate([lstm_Wih1, lstm_Wih1], axis=0)
    bsum1 = (lstm_bih1 + lstm_bhh1).reshape(1, 4 * H)
    bsum2 = (lstm_bih2 + lstm_bhh2).reshape(1, 4 * H)
    return _main(g3, feat_t, lengths.reshape(B, 1).astype(jnp.int32),
                 featw_p, pad(mamba_a), pad(mamba_b), pad(mamba_c),
                 pad(mamba_d), ev_p, wih1_s, lstm_Whh1, bsum1,
                 lstm_Wih2, lstm_Whh2, bsum2, cls_W, cls_b)


# speedup vs baseline: 17.7819x; 1.0548x over previous
import functools

import jax
import jax.numpy as jnp
from jax import lax
from jax.experimental import pallas as pl
from jax.experimental.pallas import tpu as pltpu
from jax.experimental.pallas import tpu_sc as plsc

B, L, D, E, G, AUX, H, NCLS = 256, 200, 32, 4, 5, 16, 128, 10
GAMMA = 0.3
NV = 784


def _table_body(rw_ref, rb_ref, w1_ref, b1_ref, bd2_ref, b2_ref, win_ref,
                tab_ref):
    t = lax.broadcasted_iota(jnp.int32, (NV, 1), 0).astype(jnp.float32) * (
        1.0 / 784.0)
    rw = rw_ref[...]
    rb = rb_ref[...]
    ls = [t * rw[:, e:e + 1] + rb[:, e:e + 1] for e in range(E)]

    sel = []
    for e in range(E):
        cnt = jnp.zeros_like(ls[e])
        for j in range(E):
            if j == e:
                continue
            beats = (ls[j] > ls[e]) if j > e else (ls[j] >= ls[e])
            cnt = cnt + jnp.where(beats, 1.0, 0.0)
        sel.append(cnt < 2.0)
    m = jnp.maximum(jnp.maximum(ls[0], ls[1]), jnp.maximum(ls[2], ls[3]))
    we = [jnp.where(sel[e], jnp.exp(ls[e] - m), 0.0) for e in range(E)]
    den = we[0] + we[1] + we[2] + we[3]
    wn = [w / den for w in we]

    acc = jnp.broadcast_to(b1_ref[...], (NV, E * 32))
    w1 = w1_ref[...]
    for g in range(G):
        gv = -2.0 + float(g)
        bg = jnp.exp(-(((t - gv) * 2.0) ** 2))
        acc = acc + bg * w1[g:g + 1, :]
    h1 = acc * jax.nn.sigmoid(acc)
    all_out = jnp.dot(h1, bd2_ref[...],
                      preferred_element_type=jnp.float32) + b2_ref[...]
    eo = jnp.zeros((NV, D), jnp.float32)
    for e in range(E):
        eo = eo + wn[e] * all_out[:, e * D:(e + 1) * D]
    xm = jnp.dot(eo, win_ref[...], preferred_element_type=jnp.float32)
    tab_ref[...] = jnp.concatenate(
        [eo, xm, jnp.zeros((NV, 2 * D), jnp.float32)], axis=1)


def _build_tables(router_W, router_b, kan_W1, kan_b1, kan_W2, kan_b2,
                  mamba_Win):
    w1r = kan_W1.transpose(1, 0, 2).reshape(G, E * 32)
    b1r = kan_b1.reshape(1, E * 32)
    bd2 = jax.scipy.linalg.block_diag(*[kan_W2[e] for e in range(E)])
    b2r = kan_b2.reshape(1, E * D)
    return pl.pallas_call(
        _table_body,
        out_shape=jax.ShapeDtypeStruct((NV, 4 * D), jnp.float32),
    )(router_W, router_b.reshape(1, E), w1r, b1r, bd2, b2r, mamba_Win)


_NW = 32
_CH = 80
_GRP = 5


def _make_gather_body(bp, nch):
    def _gather_body(idx_hbm, tab_hbm, g_hbm, idx_v, rows_v, sem_g, sem_o):
        wid = lax.axis_index("s") * 2 + lax.axis_index("c")
        base = wid * bp
        pltpu.sync_copy(idx_hbm.at[pl.ds(base, bp)], idx_v)

        def group(grp, carry):
            offs = [grp * _GRP * _CH + j * _CH for j in range(_GRP)]
            bufs = [rows_v.at[j] for j in range(_GRP)]
            @pl.when(grp > 0)
            def _():
                for j in range(_GRP):
                    pltpu.make_async_copy(
                        bufs[j], g_hbm.at[pl.ds(base + offs[j], _CH)],
                        sem_o).wait()
            for j in range(_GRP):
                pltpu.async_copy(tab_hbm.at[idx_v.at[pl.ds(offs[j], _CH)]],
                                 bufs[j], sem_g)
            for j in range(_GRP):
                pltpu.make_async_copy(
                    tab_hbm.at[idx_v.at[pl.ds(offs[j], _CH)]], bufs[j],
                    sem_g).wait()
            for j in range(_GRP):
                pltpu.async_copy(bufs[j],
                                 g_hbm.at[pl.ds(base + offs[j], _CH)], sem_o)
            return carry

        lax.fori_loop(0, nch // _GRP, group, 0)
        for j in range(_GRP):
            pltpu.make_async_copy(rows_v.at[j],
                                  g_hbm.at[pl.ds(base, _CH)], sem_o).wait()

    return _gather_body


def _gather(idx, table, ntok):
    bp = ntok // _NW
    mesh = plsc.VectorSubcoreMesh(core_axis_name="c", subcore_axis_name="s")
    f = functools.partial(
        pl.kernel,
        mesh=mesh,
        out_type=jax.ShapeDtypeStruct((ntok, 4 * D), jnp.float32),
        scratch_types=[pltpu.VMEM((bp,), jnp.int32),
                       pltpu.VMEM((_GRP, _CH, 4 * D), jnp.float32),
                       pltpu.SemaphoreType.DMA,
                       pltpu.SemaphoreType.DMA],
    )(_make_gather_body(bp, bp // _CH))
    return f(idx, table)


def _sig(x):
    return 0.5 * jnp.tanh(0.5 * x) + 0.5


_CS = 10
_HL = L // 2
_HC = _HL // _CS


def _main_body(g_ref, feat_ref, len_ref,
               s_in, h1_in, c1_in, h2_in, c2_in,
               featw_ref, ma_ref, mb_ref, mc_ref, md_ref, e_ref,
               wih1_ref, whh1_ref, b1_ref, wih2_ref, whh2_ref, b2_ref,
               clsw_ref, clsb_ref,
               s_o, h1_o, c1_o, h2_o, c2_o, log_o,
               gbuf, fbuf, semg, semf, *, chunk0):
    def g_copy(i, slot):
        return pltpu.make_async_copy(
            g_ref.at[pl.ds(_CS * i, _CS)], gbuf.at[slot], semg.at[slot])

    def f_copy(i, slot):
        return pltpu.make_async_copy(
            feat_ref.at[pl.ds(_CS * AUX * (chunk0 + i), _CS * AUX)],
            fbuf.at[slot], semf.at[slot])

    featw = featw_ref[...]
    a_sig = jax.nn.sigmoid(ma_ref[...])
    mb = mb_ref[...]
    mc = mc_ref[...]
    md = md_ref[...]
    ev = e_ref[...]
    wih1 = wih1_ref[...]
    whh1 = whh1_ref[...]
    b1 = b1_ref[...]
    wih2 = wih2_ref[...]
    whh2 = whh2_ref[...]
    b2 = b2_ref[...]
    lens_raw = len_ref[...]
    lens = jnp.maximum(lens_raw, 1)

    def step(t, gv, ftT, carry):
        s, h1, c1, h2, c2 = carry
        g = gv[:, 0:2 * D]
        x = g + lax.dot_general(
            ftT, featw, (((0,), (0,)), ((), ())),
            preferred_element_type=jnp.float32)
        s = a_sig * s + mb * x
        y = mc * s + md * x
        emb = GAMMA * y + ev * g
        g1 = (jnp.dot(emb, wih1, preferred_element_type=jnp.float32)
              + jnp.dot(h1, whh1, preferred_element_type=jnp.float32) + b1)
        i1 = _sig(g1[:, 0:H])
        f1 = _sig(g1[:, H:2 * H])
        gg1 = jnp.tanh(g1[:, 2 * H:3 * H])
        o1 = _sig(g1[:, 3 * H:4 * H])
        c1n = f1 * c1 + i1 * gg1
        h1n = o1 * jnp.tanh(c1n)
        m = lens > t
        h1 = jnp.where(m, h1n, h1)
        c1 = jnp.where(m, c1n, c1)
        g2 = (jnp.dot(h1, wih2, preferred_element_type=jnp.float32)
              + jnp.dot(h2, whh2, preferred_element_type=jnp.float32) + b2)
        i2 = _sig(g2[:, 0:H])
        f2 = _sig(g2[:, H:2 * H])
        gg2 = jnp.tanh(g2[:, 2 * H:3 * H])
        o2 = _sig(g2[:, 3 * H:4 * H])
        c2n = f2 * c2 + i2 * gg2
        h2n = o2 * jnp.tanh(c2n)
        h2 = jnp.where(m, h2n, h2)
        c2 = jnp.where(m, c2n, c2)
        return (s, h1, c1, h2, c2)

    def chunk(i, carry):
        slot = lax.rem(i, 2)

        @pl.when(i + 1 < _HC)
        def _():
            g_copy(i + 1, 1 - slot).start()
            f_copy(i + 1, 1 - slot).start()

        g_copy(i, slot).wait()
        f_copy(i, slot).wait()
        for k in range(_CS):
            gv = gbuf[slot, k]
            ftT = fbuf[slot, pl.ds(AUX * k, AUX), :]
            carry = step(_CS * (chunk0 + i) + k, gv, ftT, carry)
        return carry

    g_copy(0, 0).start()
    f_copy(0, 0).start()
    carry0 = (s_in[...], h1_in[...], c1_in[...], h2_in[...], c2_in[...])
    s, h1, c1, h2, c2 = lax.fori_loop(0, _HC, chunk, carry0)
    s_o[...] = s
    h1_o[...] = h1
    c1_o[...] = c1
    h2_o[...] = h2
    c2_o[...] = c2
    logits = jnp.dot(h2, clsw_ref[...],
                     preferred_element_type=jnp.float32) + clsb_ref[...]
    valid = lens_raw > 0
    log_o[...] = jnp.where(valid, logits, 0.0)


def _main_half(chunk0, g3h, feat_t, lengths, state, weights):
    body = functools.partial(_main_body, chunk0=chunk0)
    sds = jax.ShapeDtypeStruct
    return pl.pallas_call(
        body,
        out_shape=(sds((B, 2 * D), jnp.float32), sds((B, H), jnp.float32),
                   sds((B, H), jnp.float32), sds((B, H), jnp.float32),
                   sds((B, H), jnp.float32), sds((B, NCLS), jnp.float32)),
        in_specs=[pl.BlockSpec(memory_space=pl.ANY)] * 2
        + [pl.BlockSpec(memory_space=pltpu.MemorySpace.VMEM)] * 20,
        scratch_shapes=[
            pltpu.VMEM((2, _CS, B, 4 * D), jnp.float32),
            pltpu.VMEM((2, _CS * AUX, B), jnp.float32),
            pltpu.SemaphoreType.DMA((2,)),
            pltpu.SemaphoreType.DMA((2,)),
        ],
        compiler_params=pltpu.CompilerParams(
            vmem_limit_bytes=100 * 1024 * 1024),
    )(g3h, feat_t, lengths, *state, *weights)


def kernel(events, features, lengths, router_W, router_b, kan_W1, kan_b1,
           kan_W2, kan_b2, feat_W, mamba_Win, mamba_a, mamba_b, mamba_c,
           mamba_d, lstm_Wih1, lstm_Whh1, lstm_bih1, lstm_bhh1,
           lstm_Wih2, lstm_Whh2, lstm_bih2, lstm_bhh2, cls_W, cls_b):
    table = _build_tables(router_W, router_b, kan_W1, kan_b1,
                          kan_W2, kan_b2, mamba_Win)
    idx = events.astype(jnp.int32).T.reshape(B * L)
    g3a = _gather(idx[:B * _HL], table, B * _HL).reshape(_HL, B, 4 * D)
    g3b = _gather(idx[B * _HL:], table, B * _HL).reshape(_HL, B, 4 * D)
    feat_t = features.reshape(B, L * AUX).T
    zc = jnp.zeros((1, D), jnp.float32)
    pad = lambda v: jnp.concatenate([zc, v.reshape(1, D)], axis=1)
    featw_p = jnp.concatenate(
        [jnp.zeros((AUX, D), jnp.float32), feat_W], axis=1)
    ev_p = jnp.concatenate(
        [jnp.full((1, D), 1.0 - GAMMA, jnp.float32), zc], axis=1)
    wih1_s = jnp.concatenate([lstm_Wih1, lstm_Wih1], axis=0)
    bsum1 = (lstm_bih1 + lstm_bhh1).reshape(1, 4 * H)
    bsum2 = (lstm_bih2 + lstm_bhh2).reshape(1, 4 * H)
    weights = (featw_p, pad(mamba_a), pad(mamba_b), pad(mamba_c),
               pad(mamba_d), ev_p, wih1_s, lstm_Whh1, bsum1,
               lstm_Wih2, lstm_Whh2, bsum2, cls_W, cls_b.reshape(1, NCLS))
    lens2d = lengths.reshape(B, 1).astype(jnp.int32)
    zs = jnp.zeros((B, 2 * D), jnp.float32)
    zh = jnp.zeros((B, H), jnp.float32)
    st = _main_half(0, g3a, feat_t, lens2d, (zs, zh, zh, zh, zh), weights)
    out = _main_half(_HC, g3b, feat_t, lens2d, st[:5], weights)
    return out[5]
